# baseline (speedup 1.0000x reference)
import jax
import jax.numpy as jnp
from jax.experimental import pallas as pl
from jax.experimental.pallas import tpu as pltpu

N_TOKENS = 32768
HIDDEN = 768
N_EXPERTS = 64
TOP_K = 2
BLK = 512
NBUF = 8
GRID = N_TOKENS // BLK


def _router_kernel(x_hbm, w_ref, idx_ref, tw_ref, loss_ref, frac_ref,
                   probs_ref, xbuf, psum_ref, sems):
    i = pl.program_id(0)
    nsteps = pl.num_programs(0)

    @pl.when(i == 0)
    def _prologue():
        for b in range(NBUF):
            pltpu.make_async_copy(
                x_hbm.at[pl.ds(b * BLK, BLK), :], xbuf.at[b], sems.at[b]
            ).start()

    slot = jax.lax.rem(i, NBUF)
    pltpu.make_async_copy(
        x_hbm.at[pl.ds(i * BLK, BLK), :], xbuf.at[slot], sems.at[slot]
    ).wait()

    logits = jax.lax.dot_general(
        xbuf[slot], w_ref[...], (((1,), (1,)), ((), ())),
        preferred_element_type=jnp.float32)

    @pl.when(i + NBUF < nsteps)
    def _prefetch():
        pltpu.make_async_copy(
            x_hbm.at[pl.ds((i + NBUF) * BLK, BLK), :], xbuf.at[slot],
            sems.at[slot]
        ).start()

    cols = jax.lax.broadcasted_iota(jnp.int32, logits.shape, 1)
    m1 = jnp.max(logits, axis=1, keepdims=True)
    idx1 = jnp.min(jnp.where(logits == m1, cols, N_EXPERTS), axis=1,
                   keepdims=True)
    masked = jnp.where(cols == idx1, -jnp.inf, logits)
    m2 = jnp.max(masked, axis=1, keepdims=True)
    idx2 = jnp.min(jnp.where(masked == m2, cols, N_EXPERTS), axis=1,
                   keepdims=True)

    e = jnp.exp(logits - m1)
    probs = e / jnp.sum(e, axis=1, keepdims=True)
    probs_ref[...] = probs

    w1 = 1.0 / (1.0 + jnp.exp(m2 - m1))
    tw_ref[...] = jnp.concatenate([w1, 1.0 - w1], axis=1)
    idx_ref[...] = jnp.concatenate([idx1, idx2], axis=1)

    onehot = ((cols == idx1).astype(jnp.float32)
              + (cols == idx2).astype(jnp.float32))
    cnt = jnp.sum(onehot, axis=0, keepdims=True)
    ps = jnp.sum(probs, axis=0, keepdims=True)

    @pl.when(i == 0)
    def _init():
        frac_ref[...] = jnp.zeros_like(frac_ref)
        psum_ref[...] = jnp.zeros_like(psum_ref)

    frac_ref[...] += cnt
    psum_ref[...] += ps

    @pl.when(i == nsteps - 1)
    def _fin():
        counts = frac_ref[...]
        inv_n = 1.0 / N_TOKENS
        loss_ref[...] = (N_EXPERTS * inv_n * inv_n) * jnp.sum(
            counts * psum_ref[...], keepdims=True)
        frac_ref[...] = counts * inv_n


def kernel(x, W):
    idx, tw, loss, frac, probs = pl.pallas_call(
        _router_kernel,
        grid=(GRID,),
        in_specs=[
            pl.BlockSpec(memory_space=pl.ANY),
            pl.BlockSpec((N_EXPERTS, HIDDEN), lambda i: (0, 0)),
        ],
        out_specs=[
            pl.BlockSpec((BLK, TOP_K), lambda i: (i, 0)),
            pl.BlockSpec((BLK, TOP_K), lambda i: (i, 0)),
            pl.BlockSpec((1, 1), lambda i: (0, 0)),
            pl.BlockSpec((1, N_EXPERTS), lambda i: (0, 0)),
            pl.BlockSpec((BLK, N_EXPERTS), lambda i: (i, 0)),
        ],
        out_shape=[
            jax.ShapeDtypeStruct((N_TOKENS, TOP_K), jnp.int32),
            jax.ShapeDtypeStruct((N_TOKENS, TOP_K), jnp.float32),
            jax.ShapeDtypeStruct((1, 1), jnp.float32),
            jax.ShapeDtypeStruct((1, N_EXPERTS), jnp.float32),
            jax.ShapeDtypeStruct((N_TOKENS, N_EXPERTS), jnp.float32),
        ],
        scratch_shapes=[
            pltpu.VMEM((NBUF, BLK, HIDDEN), jnp.float32),
            pltpu.VMEM((1, N_EXPERTS), jnp.float32),
            pltpu.SemaphoreType.DMA((NBUF,)),
        ],
    )(x, W)
    return idx, tw, loss[0, 0], frac[0], probs


# speedup vs baseline: 1.1862x; 1.1862x over previous
import jax
import jax.numpy as jnp
from jax.experimental import pallas as pl
from jax.experimental.pallas import tpu as pltpu

N_TOKENS = 32768
HIDDEN = 768
N_EXPERTS = 64
TOP_K = 2
BLK = 2048
NBUF = 4
GRID = N_TOKENS // BLK


def _router_kernel(x_hbm, w_ref, idx_ref, tw_ref, loss_ref, frac_ref,
                   probs_ref, xbuf, psum_ref, sems):
    i = pl.program_id(0)
    nsteps = pl.num_programs(0)

    @pl.when(i == 0)
    def _prologue():
        for b in range(NBUF):
            pltpu.make_async_copy(
                x_hbm.at[pl.ds(b * BLK, BLK), :], xbuf.at[b], sems.at[b]
            ).start()

    slot = jax.lax.rem(i, NBUF)
    pltpu.make_async_copy(
        x_hbm.at[pl.ds(i * BLK, BLK), :], xbuf.at[slot], sems.at[slot]
    ).wait()

    logits = jax.lax.dot_general(
        xbuf[slot], w_ref[...], (((1,), (1,)), ((), ())),
        preferred_element_type=jnp.float32)

    @pl.when(i + NBUF < nsteps)
    def _prefetch():
        pltpu.make_async_copy(
            x_hbm.at[pl.ds((i + NBUF) * BLK, BLK), :], xbuf.at[slot],
            sems.at[slot]
        ).start()

    cols = jax.lax.broadcasted_iota(jnp.int32, logits.shape, 1)
    m1 = jnp.max(logits, axis=1, keepdims=True)
    idx1 = jnp.min(jnp.where(logits == m1, cols, N_EXPERTS), axis=1,
                   keepdims=True)
    masked = jnp.where(cols == idx1, -jnp.inf, logits)
    m2 = jnp.max(masked, axis=1, keepdims=True)
    idx2 = jnp.min(jnp.where(masked == m2, cols, N_EXPERTS), axis=1,
                   keepdims=True)

    e = jnp.exp(logits - m1)
    probs = e / jnp.sum(e, axis=1, keepdims=True)
    probs_ref[...] = probs

    w1 = 1.0 / (1.0 + jnp.exp(m2 - m1))
    tw_ref[...] = jnp.concatenate([w1, 1.0 - w1], axis=1)
    idx_ref[...] = jnp.concatenate([idx1, idx2], axis=1)

    onehot = ((cols == idx1).astype(jnp.float32)
              + (cols == idx2).astype(jnp.float32))
    cnt = jnp.sum(onehot, axis=0, keepdims=True)
    ps = jnp.sum(probs, axis=0, keepdims=True)

    @pl.when(i == 0)
    def _init():
        frac_ref[...] = jnp.zeros_like(frac_ref)
        psum_ref[...] = jnp.zeros_like(psum_ref)

    frac_ref[...] += cnt
    psum_ref[...] += ps

    @pl.when(i == nsteps - 1)
    def _fin():
        counts = frac_ref[...]
        inv_n = 1.0 / N_TOKENS
        loss_ref[...] = (N_EXPERTS * inv_n * inv_n) * jnp.sum(
            counts * psum_ref[...], keepdims=True)
        frac_ref[...] = counts * inv_n


def kernel(x, W):
    idx, tw, loss, frac, probs = pl.pallas_call(
        _router_kernel,
        grid=(GRID,),
        in_specs=[
            pl.BlockSpec(memory_space=pl.ANY),
            pl.BlockSpec((N_EXPERTS, HIDDEN), lambda i: (0, 0)),
        ],
        out_specs=[
            pl.BlockSpec((BLK, TOP_K), lambda i: (i, 0)),
            pl.BlockSpec((BLK, TOP_K), lambda i: (i, 0)),
            pl.BlockSpec((1, 1), lambda i: (0, 0)),
            pl.BlockSpec((1, N_EXPERTS), lambda i: (0, 0)),
            pl.BlockSpec((BLK, N_EXPERTS), lambda i: (i, 0)),
        ],
        out_shape=[
            jax.ShapeDtypeStruct((N_TOKENS, TOP_K), jnp.int32),
            jax.ShapeDtypeStruct((N_TOKENS, TOP_K), jnp.float32),
            jax.ShapeDtypeStruct((1, 1), jnp.float32),
            jax.ShapeDtypeStruct((1, N_EXPERTS), jnp.float32),
            jax.ShapeDtypeStruct((N_TOKENS, N_EXPERTS), jnp.float32),
        ],
        scratch_shapes=[
            pltpu.VMEM((NBUF, BLK, HIDDEN), jnp.float32),
            pltpu.VMEM((1, N_EXPERTS), jnp.float32),
            pltpu.SemaphoreType.DMA((NBUF,)),
        ],
    )(x, W)
    return idx, tw, loss[0, 0], frac[0], probs
